# baseline (speedup 1.0000x reference)
import functools

import jax
import jax.numpy as jnp
from jax import lax
from jax.experimental import pallas as pl
from jax.experimental.pallas import tpu as pltpu
from jax.experimental.pallas import tpu_sc as plsc


def _sc_gather(input_ids, table_pairs, V, D):
    B = input_ids.shape[0]
    L = 16
    info = plsc.get_sparse_core_info()
    NC, NS = info.num_cores, info.num_subcores
    NW = NC * NS
    assert B % (8 * NW) == 0 and D % L == 0
    b_per_w = B // NW
    n_out = b_per_w * D // 128

    mesh = plsc.VectorSubcoreMesh(core_axis_name="c", subcore_axis_name="s")

    @functools.partial(
        pl.kernel,
        mesh=mesh,
        compiler_params=pltpu.CompilerParams(needs_layout_passes=False),
        out_type=jax.ShapeDtypeStruct((B * D // 128, 128), jnp.float32),
        scratch_types=[
            pltpu.VMEM((b_per_w + 8,), jnp.int32),
            pltpu.VMEM((b_per_w,), jnp.int32),
            pltpu.VMEM((b_per_w, 2 * D), jnp.float32),
            pltpu.VMEM((n_out, 128), jnp.float32),
            pltpu.SemaphoreType.DMA,
        ],
    )
    def gather_kernel(idx_hbm, table_hbm, out_hbm, idx_v, idx2_v, pairs_v,
                      rows_v, sem):
        wid = lax.axis_index("s") * NC + lax.axis_index("c")
        base = wid * b_per_w
        pltpu.sync_copy(idx_hbm.at[pl.ds(base, b_per_w)], idx_v.at[pl.ds(8, b_per_w)])
        for g in range(b_per_w // L):
            idx2_v[pl.ds(g * L, L)] = lax.shift_right_logical(
                idx_v[pl.ds(8 + g * L, L)], 1
            )
        pltpu.async_copy(table_hbm.at[idx2_v], pairs_v, sem).wait()
        for j in range(b_per_w):
            rj = plsc.load_gather(idx_v, [jnp.full((L,), j + 8, jnp.int32)])
            odd = lax.eq(rj & jnp.full((L,), 1, jnp.int32),
                         jnp.full((L,), 1, jnp.int32))
            for k in range(D // L):
                lo = pairs_v[j, pl.ds(k * L, L)]
                hi = pairs_v[j, pl.ds(D + k * L, L)]
                rows_v[(j * D + k * L) // 128,
                       pl.ds((j * D + k * L) % 128, L)] = jnp.where(odd, hi, lo)
        pltpu.sync_copy(rows_v, out_hbm.at[pl.ds(wid * n_out, n_out)])

    return gather_kernel(input_ids, table_pairs)


def _tc_project_t(emb, w_t, lin_b):
    B, D = emb.shape
    V = w_t.shape[1]
    BV = 2048

    def body(w_ref, emb_ref, b_ref, out_ref):
        out_ref[...] = lax.dot_general(
            w_ref[...], emb_ref[...],
            (((0,), (1,)), ((), ())),
            preferred_element_type=jnp.float32,
        ) + b_ref[...]

    return pl.pallas_call(
        body,
        grid=(pl.cdiv(V, BV),),
        in_specs=[
            pl.BlockSpec((D, BV), lambda i: (0, i)),
            pl.BlockSpec((B, D), lambda i: (0, 0)),
            pl.BlockSpec((BV, 1), lambda i: (i, 0)),
        ],
        out_specs=pl.BlockSpec((BV, B), lambda i: (i, 0)),
        out_shape=jax.ShapeDtypeStruct((V, B), jnp.float32),
    )(w_t, emb, lin_b.reshape(V, 1))


def kernel(input_ids, emb_table, lin_w, lin_b):
    V, D = emb_table.shape
    emb_2d = _sc_gather(
        input_ids.astype(jnp.int32), emb_table.reshape(V // 2, 2 * D), V, D
    )
    emb = emb_2d.reshape(input_ids.shape[0], D)
    return _tc_project_t(emb, lin_w.T, lin_b).T


# speedup vs baseline: 1.1305x; 1.1305x over previous
import functools

import jax
import jax.numpy as jnp
from jax import lax
from jax.experimental import pallas as pl
from jax.experimental.pallas import tpu as pltpu
from jax.experimental.pallas import tpu_sc as plsc


def _sc_gather(input_ids, table_flat, V, D):
    B = input_ids.shape[0]
    L = 16
    info = plsc.get_sparse_core_info()
    NC, NS = info.num_cores, info.num_subcores
    NW = NC * NS
    assert B % (8 * NW) == 0 and D % L == 0
    b_per_w = B // NW
    n_flat = b_per_w * D
    n_chunks = n_flat // 128

    mesh = plsc.VectorSubcoreMesh(core_axis_name="c", subcore_axis_name="s")

    @functools.partial(
        pl.kernel,
        mesh=mesh,
        compiler_params=pltpu.CompilerParams(needs_layout_passes=False),
        out_type=jax.ShapeDtypeStruct((B * D // 128, 128), jnp.float32),
        scratch_types=[
            pltpu.VMEM((b_per_w + 8,), jnp.int32),
            pltpu.VMEM((n_chunks, 128), jnp.int32),
            pltpu.VMEM((n_chunks, 128), jnp.float32),
            pltpu.SemaphoreType.DMA,
        ],
    )
    def gather_kernel(idx_hbm, table_hbm, out_hbm, idx_v, flat_v, rows_v, sem):
        wid = lax.axis_index("s") * NC + lax.axis_index("c")
        base = wid * b_per_w
        pltpu.sync_copy(idx_hbm.at[pl.ds(base, b_per_w)], idx_v.at[pl.ds(8, b_per_w)])
        lane = lax.iota(jnp.int32, L)
        for j in range(b_per_w):
            rj = plsc.load_gather(idx_v, [jnp.full((L,), j + 8, jnp.int32)])
            for k in range(D // L):
                p = j * D + k * L
                flat_v[p // 128, pl.ds(p % 128, L)] = (lane + k * L) * V + rj
        for c in range(n_chunks):
            pltpu.async_copy(
                table_hbm.at[flat_v.at[c]],
                rows_v.at[c],
                sem,
            ).start()
        for c in range(n_chunks):
            pltpu.async_copy(
                table_hbm.at[flat_v.at[c]],
                rows_v.at[c],
                sem,
            ).wait()
        pltpu.sync_copy(rows_v, out_hbm.at[pl.ds(wid * n_chunks, n_chunks)])

    return gather_kernel(input_ids, table_flat)


def _tc_project_t(emb, w_t, lin_b):
    B, D = emb.shape
    V = w_t.shape[1]
    BV = 2048

    def body(w_ref, emb_ref, b_ref, out_ref):
        out_ref[...] = lax.dot_general(
            w_ref[...], emb_ref[...],
            (((0,), (1,)), ((), ())),
            preferred_element_type=jnp.float32,
        ) + b_ref[...]

    return pl.pallas_call(
        body,
        grid=(pl.cdiv(V, BV),),
        in_specs=[
            pl.BlockSpec((D, BV), lambda i: (0, i)),
            pl.BlockSpec((B, D), lambda i: (0, 0)),
            pl.BlockSpec((BV, 1), lambda i: (i, 0)),
        ],
        out_specs=pl.BlockSpec((BV, B), lambda i: (i, 0)),
        out_shape=jax.ShapeDtypeStruct((V, B), jnp.float32),
    )(w_t, emb, lin_b.reshape(V, 1))


def kernel(input_ids, emb_table, lin_w, lin_b):
    V, D = emb_table.shape
    emb_2d = _sc_gather(
        input_ids.astype(jnp.int32),
        lax.reshape(emb_table, (V * D,), dimensions=(1, 0)), V, D
    )
    emb = emb_2d.reshape(input_ids.shape[0], D)
    return _tc_project_t(emb, lin_w.T, lin_b).T
